# baseline (speedup 1.0000x reference)
import functools

import jax
import jax.numpy as jnp
from jax import lax
from jax.experimental import pallas as pl
from jax.experimental.pallas import tpu as pltpu
from jax.experimental.pallas import tpu_sc as plsc

NC = 2
NS = 16
NW = NC * NS

BATCH = 16384
DIM = 32
NROWS = 1000000
CHUNK = 128
CHUNKS_PER_W = BATCH // (NW * CHUNK)

LB = 1024
NG = (NROWS + LB - 1) // LB
OUTR = NG * (LB // 4)


def _relayout_body(x_ref, o_ref):
    x = x_ref[...]
    for p in range(4):
        o_ref[:, DIM * p:DIM * (p + 1)] = x[:, 256 * p:256 * (p + 1)].T


def _relayout(tT):
    return pl.pallas_call(
        _relayout_body,
        grid=(NG,),
        in_specs=[pl.BlockSpec((DIM, LB), lambda g: (0, g))],
        out_specs=pl.BlockSpec((LB // 4, 128), lambda g: (g, 0)),
        out_shape=jax.ShapeDtypeStruct((OUTR, 128), jnp.float32),
    )(tT)


def _sc_gather1(idx2, tbl):
    mesh = plsc.VectorSubcoreMesh(core_axis_name="c", subcore_axis_name="s")

    @functools.partial(
        pl.kernel,
        out_type=jax.ShapeDtypeStruct((BATCH, 128), jnp.float32),
        mesh=mesh,
        scratch_types=[
            pltpu.VMEM((CHUNKS_PER_W, 128), jnp.int32),
            pltpu.VMEM((CHUNK, 128), jnp.float32),
            pltpu.VMEM((CHUNK, 128), jnp.float32),
            pltpu.SemaphoreType.DMA,
            pltpu.SemaphoreType.DMA,
        ],
    )
    def k(tbl_hbm, idx_hbm, out, idx_v, buf0, buf1, sem0, sem1):
        wid = lax.axis_index("s") * NC + lax.axis_index("c")
        row0 = wid * CHUNKS_PER_W
        pltpu.sync_copy(idx_hbm.at[pl.ds(row0, CHUNKS_PER_W)], idx_v)
        bufs = (buf0, buf1)
        sems = (sem0, sem1)
        descs = [None, None]
        for j in range(CHUNKS_PER_W):
            descs[j % 2] = pltpu.async_copy(
                tbl_hbm.at[idx_v.at[j]], bufs[j % 2], sems[j % 2])
            if j >= 1:
                p = (j - 1) % 2
                descs[p].wait()
                pltpu.sync_copy(bufs[p],
                                out.at[pl.ds((row0 + j - 1) * CHUNK, CHUNK)])
        last = (CHUNKS_PER_W - 1) % 2
        descs[last].wait()
        pltpu.sync_copy(
            bufs[last],
            out.at[pl.ds((row0 + CHUNKS_PER_W - 1) * CHUNK, CHUNK)])

    return k(tbl, idx2)


BLK = 2048


def _sel(x, m):
    r = jnp.where(m == 0, x[:, 0 * DIM:1 * DIM], x[:, 1 * DIM:2 * DIM])
    r = jnp.where(m == 2, x[:, 2 * DIM:3 * DIM], r)
    return jnp.where(m == 3, x[:, 3 * DIM:4 * DIM], r)


def _tc_body(us_ref, is_ref, ug_ref, ig_ref, um_ref, im_ref,
             w0u_ref, w0i_ref, b0_ref, w1_ref, b1_ref, w2_ref, b2_ref,
             whg_ref, whh_ref, bh_ref, o_ref):
    f32 = jnp.float32
    mu = us_ref[...]
    mi = is_ref[...]
    ug = _sel(ug_ref[...], mu)
    ig = _sel(ig_ref[...], mi)
    um = _sel(um_ref[...], mu)
    im = _sel(im_ref[...], mi)
    h = jnp.dot(um, w0u_ref[...], preferred_element_type=f32)
    h = h + jnp.dot(im, w0i_ref[...], preferred_element_type=f32)
    h = jnp.maximum(h + b0_ref[...], 0.0)
    h = jnp.maximum(jnp.dot(h, w1_ref[...], preferred_element_type=f32)
                    + b1_ref[...], 0.0)
    h = jnp.maximum(jnp.dot(h, w2_ref[...], preferred_element_type=f32)
                    + b2_ref[...], 0.0)
    gmf = ug * ig
    o_ref[...] = (jnp.dot(gmf, whg_ref[...], preferred_element_type=f32)
                  + jnp.dot(h, whh_ref[...], preferred_element_type=f32)
                  + bh_ref[...])


def _tc_dense(us, is_, ug, ig, um, im, W0, b0, W1, b1, W2, b2, Wh, bh):
    w0u = W0[:, :DIM].T
    w0i = W0[:, DIM:].T
    w1 = W1.T
    w2 = W2.T
    whg = Wh[:, :DIM].T
    whh = Wh[:, DIM:].T
    b0r = b0.reshape(1, -1)
    b1r = b1.reshape(1, -1)
    b2r = b2.reshape(1, -1)
    bhr = bh.reshape(1, 1)

    n_blk = BATCH // BLK
    row_spec = pl.BlockSpec((BLK, 128), lambda b: (b, 0))
    sel_spec = pl.BlockSpec((BLK, 1), lambda b: (b, 0))

    def w_spec(shape):
        return pl.BlockSpec(shape, lambda b: (0, 0))

    out = pl.pallas_call(
        _tc_body,
        grid=(n_blk,),
        in_specs=[
            sel_spec, sel_spec,
            row_spec, row_spec, row_spec, row_spec,
            w_spec(w0u.shape), w_spec(w0i.shape), w_spec(b0r.shape),
            w_spec(w1.shape), w_spec(b1r.shape),
            w_spec(w2.shape), w_spec(b2r.shape),
            w_spec(whg.shape), w_spec(whh.shape), w_spec(bhr.shape),
        ],
        out_specs=pl.BlockSpec((BLK, 1), lambda b: (b, 0)),
        out_shape=jax.ShapeDtypeStruct((BATCH, 1), jnp.float32),
    )(us, is_, ug, ig, um, im, w0u, w0i, b0r, w1, b1r, w2, b2r,
      whg, whh, bhr)
    return out[:, 0]


def kernel(u, i, user_gmf, item_gmf, user_mlp, item_mlp,
           W0, b0, W1, b1, W2, b2, Wh, bh):
    u2 = (((u >> 10) << 8) | (u & 255)).reshape(BATCH // 128, 128)
    i2 = (((i >> 10) << 8) | (i & 255)).reshape(BATCH // 128, 128)
    ug = _sc_gather1(u2, _relayout(user_gmf.T))
    ig = _sc_gather1(i2, _relayout(item_gmf.T))
    um = _sc_gather1(u2, _relayout(user_mlp.T))
    im = _sc_gather1(i2, _relayout(item_mlp.T))
    us = ((u >> 8) & 3).reshape(BATCH, 1)
    is_ = ((i >> 8) & 3).reshape(BATCH, 1)
    return _tc_dense(us, is_, ug, ig, um, im,
                     W0, b0, W1, b1, W2, b2, Wh, bh)


# speedup vs baseline: 3.0601x; 3.0601x over previous
import functools

import jax
import jax.numpy as jnp
from jax import lax
from jax.experimental import pallas as pl
from jax.experimental.pallas import tpu as pltpu
from jax.experimental.pallas import tpu_sc as plsc

NC = 2
NS = 16
NW = NC * NS

BATCH = 16384
DIM = 32
NROWS = 1000000
CHUNK = 128
CHUNKS_PER_W = BATCH // (NW * CHUNK)

LB = 4096
NG = (NROWS + LB - 1) // LB
OUTR = NG * (LB // 4)


def _relayout_body(x_ref, o_ref):
    x = x_ref[...]
    for s in range(LB // 512):
        stacked = jnp.concatenate(
            [x[:, 512 * s + 128 * q:512 * s + 128 * (q + 1)]
             for q in range(4)], axis=0)
        o_ref[128 * s:128 * (s + 1), :] = stacked.T


def _relayout(tT):
    return pl.pallas_call(
        _relayout_body,
        grid=(NG,),
        in_specs=[pl.BlockSpec((DIM, LB), lambda g: (0, g))],
        out_specs=pl.BlockSpec((LB // 4, 128), lambda g: (g, 0)),
        out_shape=jax.ShapeDtypeStruct((OUTR, 128), jnp.float32),
    )(tT)


def _sc_gather1(idx2, tbl):
    mesh = plsc.VectorSubcoreMesh(core_axis_name="c", subcore_axis_name="s")

    @functools.partial(
        pl.kernel,
        out_type=jax.ShapeDtypeStruct((BATCH, 128), jnp.float32),
        mesh=mesh,
        scratch_types=[
            pltpu.VMEM((CHUNKS_PER_W, 128), jnp.int32),
            pltpu.VMEM((CHUNK, 128), jnp.float32),
            pltpu.VMEM((CHUNK, 128), jnp.float32),
            pltpu.SemaphoreType.DMA,
            pltpu.SemaphoreType.DMA,
        ],
    )
    def k(tbl_hbm, idx_hbm, out, idx_v, buf0, buf1, sem0, sem1):
        wid = lax.axis_index("s") * NC + lax.axis_index("c")
        row0 = wid * CHUNKS_PER_W
        pltpu.sync_copy(idx_hbm.at[pl.ds(row0, CHUNKS_PER_W)], idx_v)
        bufs = (buf0, buf1)
        sems = (sem0, sem1)
        descs = [None, None]
        for j in range(CHUNKS_PER_W):
            descs[j % 2] = pltpu.async_copy(
                tbl_hbm.at[idx_v.at[j]], bufs[j % 2], sems[j % 2])
            if j >= 1:
                p = (j - 1) % 2
                descs[p].wait()
                pltpu.sync_copy(bufs[p],
                                out.at[pl.ds((row0 + j - 1) * CHUNK, CHUNK)])
        last = (CHUNKS_PER_W - 1) % 2
        descs[last].wait()
        pltpu.sync_copy(
            bufs[last],
            out.at[pl.ds((row0 + CHUNKS_PER_W - 1) * CHUNK, CHUNK)])

    return k(tbl, idx2)


BLK = 2048


def _sel(x, m):
    r = jnp.where(m == 0, x[:, 0 * DIM:1 * DIM], x[:, 1 * DIM:2 * DIM])
    r = jnp.where(m == 2, x[:, 2 * DIM:3 * DIM], r)
    return jnp.where(m == 3, x[:, 3 * DIM:4 * DIM], r)


def _tc_body(us_ref, is_ref, ug_ref, ig_ref, um_ref, im_ref,
             w0u_ref, w0i_ref, b0_ref, w1_ref, b1_ref, w2_ref, b2_ref,
             whg_ref, whh_ref, bh_ref, o_ref):
    f32 = jnp.float32
    mu = us_ref[...]
    mi = is_ref[...]
    ug = _sel(ug_ref[...], mu)
    ig = _sel(ig_ref[...], mi)
    um = _sel(um_ref[...], mu)
    im = _sel(im_ref[...], mi)
    h = jnp.dot(um, w0u_ref[...], preferred_element_type=f32)
    h = h + jnp.dot(im, w0i_ref[...], preferred_element_type=f32)
    h = jnp.maximum(h + b0_ref[...], 0.0)
    h = jnp.maximum(jnp.dot(h, w1_ref[...], preferred_element_type=f32)
                    + b1_ref[...], 0.0)
    h = jnp.maximum(jnp.dot(h, w2_ref[...], preferred_element_type=f32)
                    + b2_ref[...], 0.0)
    gmf = ug * ig
    o_ref[...] = (jnp.dot(gmf, whg_ref[...], preferred_element_type=f32)
                  + jnp.dot(h, whh_ref[...], preferred_element_type=f32)
                  + bh_ref[...])


def _tc_dense(us, is_, ug, ig, um, im, W0, b0, W1, b1, W2, b2, Wh, bh):
    w0u = W0[:, :DIM].T
    w0i = W0[:, DIM:].T
    w1 = W1.T
    w2 = W2.T
    whg = Wh[:, :DIM].T
    whh = Wh[:, DIM:].T
    b0r = b0.reshape(1, -1)
    b1r = b1.reshape(1, -1)
    b2r = b2.reshape(1, -1)
    bhr = bh.reshape(1, 1)

    n_blk = BATCH // BLK
    row_spec = pl.BlockSpec((BLK, 128), lambda b: (b, 0))
    sel_spec = pl.BlockSpec((BLK, 1), lambda b: (b, 0))

    def w_spec(shape):
        return pl.BlockSpec(shape, lambda b: (0, 0))

    out = pl.pallas_call(
        _tc_body,
        grid=(n_blk,),
        in_specs=[
            sel_spec, sel_spec,
            row_spec, row_spec, row_spec, row_spec,
            w_spec(w0u.shape), w_spec(w0i.shape), w_spec(b0r.shape),
            w_spec(w1.shape), w_spec(b1r.shape),
            w_spec(w2.shape), w_spec(b2r.shape),
            w_spec(whg.shape), w_spec(whh.shape), w_spec(bhr.shape),
        ],
        out_specs=pl.BlockSpec((BLK, 1), lambda b: (b, 0)),
        out_shape=jax.ShapeDtypeStruct((BATCH, 1), jnp.float32),
    )(us, is_, ug, ig, um, im, w0u, w0i, b0r, w1, b1r, w2, b2r,
      whg, whh, bhr)
    return out[:, 0]


def kernel(u, i, user_gmf, item_gmf, user_mlp, item_mlp,
           W0, b0, W1, b1, W2, b2, Wh, bh):
    u2 = (((u >> 9) << 7) | (u & 127)).reshape(BATCH // 128, 128)
    i2 = (((i >> 9) << 7) | (i & 127)).reshape(BATCH // 128, 128)
    ug = _sc_gather1(u2, _relayout(user_gmf.T))
    ig = _sc_gather1(i2, _relayout(item_gmf.T))
    um = _sc_gather1(u2, _relayout(user_mlp.T))
    im = _sc_gather1(i2, _relayout(item_mlp.T))
    us = ((u >> 7) & 3).reshape(BATCH, 1)
    is_ = ((i >> 7) & 3).reshape(BATCH, 1)
    return _tc_dense(us, is_, ug, ig, um, im,
                     W0, b0, W1, b1, W2, b2, Wh, bh)


# speedup vs baseline: 5.9429x; 1.9421x over previous
import functools

import jax
import jax.numpy as jnp
from jax import lax
from jax.experimental import pallas as pl
from jax.experimental.pallas import tpu as pltpu
from jax.experimental.pallas import tpu_sc as plsc

NC = 2
NS = 16
NW = NC * NS

BATCH = 16384
DIM = 32
NROWS = 1000000
CHUNK = 128
CHUNKS_PER_W = BATCH // (NW * CHUNK)

LB = 8192
NG = (NROWS + LB - 1) // LB
OUTR = NG * (LB // 4)


def _repack_one(x, o_ref):
    for s in range(LB // 512):
        stacked = jnp.concatenate(
            [x[:, 512 * s + 128 * q:512 * s + 128 * (q + 1)]
             for q in range(4)], axis=0)
        o_ref[128 * s:128 * (s + 1), :] = stacked.T


def _relayout_body(x1, x2, x3, x4, o1, o2, o3, o4):
    _repack_one(x1[...], o1)
    _repack_one(x2[...], o2)
    _repack_one(x3[...], o3)
    _repack_one(x4[...], o4)


def _relayout4(t1, t2, t3, t4):
    in_spec = pl.BlockSpec((DIM, LB), lambda g: (0, g))
    out_spec = pl.BlockSpec((LB // 4, 128), lambda g: (g, 0))
    out_t = jax.ShapeDtypeStruct((OUTR, 128), jnp.float32)
    return pl.pallas_call(
        _relayout_body,
        grid=(NG,),
        in_specs=[in_spec] * 4,
        out_specs=[out_spec] * 4,
        out_shape=[out_t] * 4,
    )(t1, t2, t3, t4)


def _sc_gather1(idx2, tbl):
    mesh = plsc.VectorSubcoreMesh(core_axis_name="c", subcore_axis_name="s")

    @functools.partial(
        pl.kernel,
        out_type=jax.ShapeDtypeStruct((BATCH, 128), jnp.float32),
        mesh=mesh,
        scratch_types=[
            pltpu.VMEM((CHUNKS_PER_W, 128), jnp.int32),
            pltpu.VMEM((CHUNK, 128), jnp.float32),
            pltpu.VMEM((CHUNK, 128), jnp.float32),
            pltpu.SemaphoreType.DMA,
            pltpu.SemaphoreType.DMA,
        ],
    )
    def k(tbl_hbm, idx_hbm, out, idx_v, buf0, buf1, sem0, sem1):
        wid = lax.axis_index("s") * NC + lax.axis_index("c")
        row0 = wid * CHUNKS_PER_W
        pltpu.sync_copy(idx_hbm.at[pl.ds(row0, CHUNKS_PER_W)], idx_v)
        bufs = (buf0, buf1)
        sems = (sem0, sem1)
        descs = [None, None]
        for j in range(CHUNKS_PER_W):
            descs[j % 2] = pltpu.async_copy(
                tbl_hbm.at[idx_v.at[j]], bufs[j % 2], sems[j % 2])
            if j >= 1:
                p = (j - 1) % 2
                descs[p].wait()
                pltpu.sync_copy(bufs[p],
                                out.at[pl.ds((row0 + j - 1) * CHUNK, CHUNK)])
        last = (CHUNKS_PER_W - 1) % 2
        descs[last].wait()
        pltpu.sync_copy(
            bufs[last],
            out.at[pl.ds((row0 + CHUNKS_PER_W - 1) * CHUNK, CHUNK)])

    return k(tbl, idx2)


BLK = 2048


def _sel(x, m):
    r = jnp.where(m == 0, x[:, 0 * DIM:1 * DIM], x[:, 1 * DIM:2 * DIM])
    r = jnp.where(m == 2, x[:, 2 * DIM:3 * DIM], r)
    return jnp.where(m == 3, x[:, 3 * DIM:4 * DIM], r)


def _tc_body(us_ref, is_ref, ug_ref, ig_ref, um_ref, im_ref,
             w0u_ref, w0i_ref, b0_ref, w1_ref, b1_ref, w2_ref, b2_ref,
             whg_ref, whh_ref, bh_ref, o_ref):
    f32 = jnp.float32
    mu = us_ref[...]
    mi = is_ref[...]
    ug = _sel(ug_ref[...], mu)
    ig = _sel(ig_ref[...], mi)
    um = _sel(um_ref[...], mu)
    im = _sel(im_ref[...], mi)
    h = jnp.dot(um, w0u_ref[...], preferred_element_type=f32)
    h = h + jnp.dot(im, w0i_ref[...], preferred_element_type=f32)
    h = jnp.maximum(h + b0_ref[...], 0.0)
    h = jnp.maximum(jnp.dot(h, w1_ref[...], preferred_element_type=f32)
                    + b1_ref[...], 0.0)
    h = jnp.maximum(jnp.dot(h, w2_ref[...], preferred_element_type=f32)
                    + b2_ref[...], 0.0)
    gmf = ug * ig
    o_ref[...] = (jnp.dot(gmf, whg_ref[...], preferred_element_type=f32)
                  + jnp.dot(h, whh_ref[...], preferred_element_type=f32)
                  + bh_ref[...])


def _tc_dense(us, is_, ug, ig, um, im, W0, b0, W1, b1, W2, b2, Wh, bh):
    w0u = W0[:, :DIM].T
    w0i = W0[:, DIM:].T
    w1 = W1.T
    w2 = W2.T
    whg = Wh[:, :DIM].T
    whh = Wh[:, DIM:].T
    b0r = b0.reshape(1, -1)
    b1r = b1.reshape(1, -1)
    b2r = b2.reshape(1, -1)
    bhr = bh.reshape(1, 1)

    n_blk = BATCH // BLK
    row_spec = pl.BlockSpec((BLK, 128), lambda b: (b, 0))
    sel_spec = pl.BlockSpec((BLK, 1), lambda b: (b, 0))

    def w_spec(shape):
        return pl.BlockSpec(shape, lambda b: (0, 0))

    out = pl.pallas_call(
        _tc_body,
        grid=(n_blk,),
        in_specs=[
            sel_spec, sel_spec,
            row_spec, row_spec, row_spec, row_spec,
            w_spec(w0u.shape), w_spec(w0i.shape), w_spec(b0r.shape),
            w_spec(w1.shape), w_spec(b1r.shape),
            w_spec(w2.shape), w_spec(b2r.shape),
            w_spec(whg.shape), w_spec(whh.shape), w_spec(bhr.shape),
        ],
        out_specs=pl.BlockSpec((BLK, 1), lambda b: (b, 0)),
        out_shape=jax.ShapeDtypeStruct((BATCH, 1), jnp.float32),
    )(us, is_, ug, ig, um, im, w0u, w0i, b0r, w1, b1r, w2, b2r,
      whg, whh, bhr)
    return out[:, 0]


def kernel(u, i, user_gmf, item_gmf, user_mlp, item_mlp,
           W0, b0, W1, b1, W2, b2, Wh, bh):
    u2 = (((u >> 9) << 7) | (u & 127)).reshape(BATCH // 128, 128)
    i2 = (((i >> 9) << 7) | (i & 127)).reshape(BATCH // 128, 128)
    ug4, ig4, um4, im4 = _relayout4(
        user_gmf.T, item_gmf.T, user_mlp.T, item_mlp.T)
    ug = _sc_gather1(u2, ug4)
    ig = _sc_gather1(i2, ig4)
    um = _sc_gather1(u2, um4)
    im = _sc_gather1(i2, im4)
    us = ((u >> 7) & 3).reshape(BATCH, 1)
    is_ = ((i >> 7) & 3).reshape(BATCH, 1)
    return _tc_dense(us, is_, ug, ig, um, im,
                     W0, b0, W1, b1, W2, b2, Wh, bh)


# speedup vs baseline: 6.0215x; 1.0132x over previous
import functools

import jax
import jax.numpy as jnp
from jax import lax
from jax.experimental import pallas as pl
from jax.experimental.pallas import tpu as pltpu
from jax.experimental.pallas import tpu_sc as plsc

NC = 2
NS = 16
NW = NC * NS

BATCH = 16384
DIM = 32
NROWS = 1000000
CHUNK = 128
CHUNKS_PER_W = BATCH // (NW * CHUNK)

LB = 16384
NG = (NROWS + LB - 1) // LB
OUTR = NG * (LB // 4)


def _repack_one(x, o_ref):
    for s in range(LB // 512):
        stacked = jnp.concatenate(
            [x[:, 512 * s + 128 * q:512 * s + 128 * (q + 1)]
             for q in range(4)], axis=0)
        o_ref[128 * s:128 * (s + 1), :] = stacked.T


def _relayout_body(x1, x2, x3, x4, o1, o2, o3, o4):
    _repack_one(x1[...], o1)
    _repack_one(x2[...], o2)
    _repack_one(x3[...], o3)
    _repack_one(x4[...], o4)


def _relayout4(t1, t2, t3, t4):
    in_spec = pl.BlockSpec((DIM, LB), lambda g: (0, g))
    out_spec = pl.BlockSpec((LB // 4, 128), lambda g: (g, 0))
    out_t = jax.ShapeDtypeStruct((OUTR, 128), jnp.float32)
    return pl.pallas_call(
        _relayout_body,
        grid=(NG,),
        in_specs=[in_spec] * 4,
        out_specs=[out_spec] * 4,
        out_shape=[out_t] * 4,
    )(t1, t2, t3, t4)


def _sc_gather1(idx2, tbl):
    mesh = plsc.VectorSubcoreMesh(core_axis_name="c", subcore_axis_name="s")

    @functools.partial(
        pl.kernel,
        out_type=jax.ShapeDtypeStruct((BATCH, 128), jnp.float32),
        mesh=mesh,
        scratch_types=[
            pltpu.VMEM((CHUNKS_PER_W, 128), jnp.int32),
            pltpu.VMEM((CHUNK, 128), jnp.float32),
            pltpu.VMEM((CHUNK, 128), jnp.float32),
            pltpu.SemaphoreType.DMA,
            pltpu.SemaphoreType.DMA,
        ],
    )
    def k(tbl_hbm, idx_hbm, out, idx_v, buf0, buf1, sem0, sem1):
        wid = lax.axis_index("s") * NC + lax.axis_index("c")
        row0 = wid * CHUNKS_PER_W
        pltpu.sync_copy(idx_hbm.at[pl.ds(row0, CHUNKS_PER_W)], idx_v)
        bufs = (buf0, buf1)
        sems = (sem0, sem1)
        descs = [None, None]
        for j in range(CHUNKS_PER_W):
            descs[j % 2] = pltpu.async_copy(
                tbl_hbm.at[idx_v.at[j]], bufs[j % 2], sems[j % 2])
            if j >= 1:
                p = (j - 1) % 2
                descs[p].wait()
                pltpu.sync_copy(bufs[p],
                                out.at[pl.ds((row0 + j - 1) * CHUNK, CHUNK)])
        last = (CHUNKS_PER_W - 1) % 2
        descs[last].wait()
        pltpu.sync_copy(
            bufs[last],
            out.at[pl.ds((row0 + CHUNKS_PER_W - 1) * CHUNK, CHUNK)])

    return k(tbl, idx2)


BLK = 4096


def _sel(x, m):
    r = jnp.where(m == 0, x[:, 0 * DIM:1 * DIM], x[:, 1 * DIM:2 * DIM])
    r = jnp.where(m == 2, x[:, 2 * DIM:3 * DIM], r)
    return jnp.where(m == 3, x[:, 3 * DIM:4 * DIM], r)


def _tc_body(us_ref, is_ref, ug_ref, ig_ref, um_ref, im_ref,
             w0u_ref, w0i_ref, b0_ref, w1_ref, b1_ref, w2_ref, b2_ref,
             whg_ref, whh_ref, bh_ref, o_ref):
    f32 = jnp.float32
    mu = us_ref[...]
    mi = is_ref[...]
    ug = _sel(ug_ref[...], mu)
    ig = _sel(ig_ref[...], mi)
    um = _sel(um_ref[...], mu)
    im = _sel(im_ref[...], mi)
    h = jnp.dot(um, w0u_ref[...], preferred_element_type=f32)
    h = h + jnp.dot(im, w0i_ref[...], preferred_element_type=f32)
    h = jnp.maximum(h + b0_ref[...], 0.0)
    h = jnp.maximum(jnp.dot(h, w1_ref[...], preferred_element_type=f32)
                    + b1_ref[...], 0.0)
    h = jnp.maximum(jnp.dot(h, w2_ref[...], preferred_element_type=f32)
                    + b2_ref[...], 0.0)
    gmf = ug * ig
    o_ref[...] = (jnp.dot(gmf, whg_ref[...], preferred_element_type=f32)
                  + jnp.dot(h, whh_ref[...], preferred_element_type=f32)
                  + bh_ref[...])


def _tc_dense(us, is_, ug, ig, um, im, W0, b0, W1, b1, W2, b2, Wh, bh):
    w0u = W0[:, :DIM].T
    w0i = W0[:, DIM:].T
    w1 = W1.T
    w2 = W2.T
    whg = Wh[:, :DIM].T
    whh = Wh[:, DIM:].T
    b0r = b0.reshape(1, -1)
    b1r = b1.reshape(1, -1)
    b2r = b2.reshape(1, -1)
    bhr = bh.reshape(1, 1)

    n_blk = BATCH // BLK
    row_spec = pl.BlockSpec((BLK, 128), lambda b: (b, 0))
    sel_spec = pl.BlockSpec((BLK, 1), lambda b: (b, 0))

    def w_spec(shape):
        return pl.BlockSpec(shape, lambda b: (0, 0))

    out = pl.pallas_call(
        _tc_body,
        grid=(n_blk,),
        in_specs=[
            sel_spec, sel_spec,
            row_spec, row_spec, row_spec, row_spec,
            w_spec(w0u.shape), w_spec(w0i.shape), w_spec(b0r.shape),
            w_spec(w1.shape), w_spec(b1r.shape),
            w_spec(w2.shape), w_spec(b2r.shape),
            w_spec(whg.shape), w_spec(whh.shape), w_spec(bhr.shape),
        ],
        out_specs=pl.BlockSpec((BLK, 1), lambda b: (b, 0)),
        out_shape=jax.ShapeDtypeStruct((BATCH, 1), jnp.float32),
    )(us, is_, ug, ig, um, im, w0u, w0i, b0r, w1, b1r, w2, b2r,
      whg, whh, bhr)
    return out[:, 0]


def kernel(u, i, user_gmf, item_gmf, user_mlp, item_mlp,
           W0, b0, W1, b1, W2, b2, Wh, bh):
    u2 = (((u >> 9) << 7) | (u & 127)).reshape(BATCH // 128, 128)
    i2 = (((i >> 9) << 7) | (i & 127)).reshape(BATCH // 128, 128)
    ug4, ig4, um4, im4 = _relayout4(
        user_gmf.T, item_gmf.T, user_mlp.T, item_mlp.T)
    ug = _sc_gather1(u2, ug4)
    ig = _sc_gather1(i2, ig4)
    um = _sc_gather1(u2, um4)
    im = _sc_gather1(i2, im4)
    us = ((u >> 7) & 3).reshape(BATCH, 1)
    is_ = ((i >> 7) & 3).reshape(BATCH, 1)
    return _tc_dense(us, is_, ug, ig, um, im,
                     W0, b0, W1, b1, W2, b2, Wh, bh)


# speedup vs baseline: 6.0987x; 1.0128x over previous
import functools

import jax
import jax.numpy as jnp
from jax import lax
from jax.experimental import pallas as pl
from jax.experimental.pallas import tpu as pltpu
from jax.experimental.pallas import tpu_sc as plsc

NC = 2
NS = 16
NW = NC * NS

BATCH = 16384
DIM = 32
NROWS = 1000000
CHUNK = 128
CHUNKS_PER_W = BATCH // (NW * CHUNK)

LB = 16384
NG = (NROWS + LB - 1) // LB
OUTR = NG * (LB // 4)


def _repack_one(x, o_ref):
    for s in range(LB // 512):
        stacked = jnp.concatenate(
            [x[:, 512 * s + 128 * q:512 * s + 128 * (q + 1)]
             for q in range(4)], axis=0)
        o_ref[128 * s:128 * (s + 1), :] = stacked.T


def _relayout_body(x1, x2, o1, o2):
    _repack_one(x1[...], o1)
    _repack_one(x2[...], o2)


def _relayout2(t1, t2):
    in_spec = pl.BlockSpec((DIM, LB), lambda g: (0, g))
    out_spec = pl.BlockSpec((LB // 4, 128), lambda g: (g, 0))
    out_t = jax.ShapeDtypeStruct((OUTR, 128), jnp.float32)
    return pl.pallas_call(
        _relayout_body,
        grid=(NG,),
        in_specs=[in_spec] * 2,
        out_specs=[out_spec] * 2,
        out_shape=[out_t] * 2,
    )(t1, t2)


def _sc_gather2(u2, i2, tblA, tblB):
    mesh = plsc.VectorSubcoreMesh(core_axis_name="c", subcore_axis_name="s")
    out_t = jax.ShapeDtypeStruct((BATCH, 128), jnp.float32)

    @functools.partial(
        pl.kernel,
        out_type=(out_t, out_t),
        mesh=mesh,
        scratch_types=[
            pltpu.VMEM((CHUNKS_PER_W, 128), jnp.int32),
            pltpu.VMEM((CHUNKS_PER_W, 128), jnp.int32),
            pltpu.VMEM((CHUNK, 128), jnp.float32),
            pltpu.VMEM((CHUNK, 128), jnp.float32),
            pltpu.VMEM((CHUNK, 128), jnp.float32),
            pltpu.VMEM((CHUNK, 128), jnp.float32),
            pltpu.SemaphoreType.DMA,
            pltpu.SemaphoreType.DMA,
            pltpu.SemaphoreType.DMA,
            pltpu.SemaphoreType.DMA,
        ],
    )
    def k(tblA_hbm, tblB_hbm, u_hbm, i_hbm, outA, outB,
          uidx, iidx, bufA0, bufA1, bufB0, bufB1, sA0, sA1, sB0, sB1):
        wid = lax.axis_index("s") * NC + lax.axis_index("c")
        row0 = wid * CHUNKS_PER_W
        pltpu.sync_copy(u_hbm.at[pl.ds(row0, CHUNKS_PER_W)], uidx)
        pltpu.sync_copy(i_hbm.at[pl.ds(row0, CHUNKS_PER_W)], iidx)
        bufsA = (bufA0, bufA1)
        bufsB = (bufB0, bufB1)
        semsA = (sA0, sA1)
        semsB = (sB0, sB1)
        descs = [None, None]

        def fire(j):
            p = j % 2
            dA = pltpu.async_copy(tblA_hbm.at[uidx.at[j]], bufsA[p], semsA[p])
            dB = pltpu.async_copy(tblB_hbm.at[iidx.at[j]], bufsB[p], semsB[p])
            descs[p] = (dA, dB)

        def drain(j):
            p = j % 2
            base = (row0 + j) * CHUNK
            descs[p][0].wait()
            pltpu.sync_copy(bufsA[p], outA.at[pl.ds(base, CHUNK)])
            descs[p][1].wait()
            pltpu.sync_copy(bufsB[p], outB.at[pl.ds(base, CHUNK)])

        fire(0)
        for j in range(1, CHUNKS_PER_W):
            fire(j)
            drain(j - 1)
        drain(CHUNKS_PER_W - 1)

    return k(tblA, tblB, u2, i2)


BLK = 4096


def _sel(x, m):
    r = jnp.where(m == 0, x[:, 0 * DIM:1 * DIM], x[:, 1 * DIM:2 * DIM])
    r = jnp.where(m == 2, x[:, 2 * DIM:3 * DIM], r)
    return jnp.where(m == 3, x[:, 3 * DIM:4 * DIM], r)


def _tc_body(us_ref, is_ref, ug_ref, ig_ref, um_ref, im_ref,
             w0u_ref, w0i_ref, b0_ref, w1_ref, b1_ref, w2_ref, b2_ref,
             whg_ref, whh_ref, bh_ref, o_ref):
    f32 = jnp.float32
    mu = us_ref[...]
    mi = is_ref[...]
    ug = _sel(ug_ref[...], mu)
    ig = _sel(ig_ref[...], mi)
    um = _sel(um_ref[...], mu)
    im = _sel(im_ref[...], mi)
    h = jnp.dot(um, w0u_ref[...], preferred_element_type=f32)
    h = h + jnp.dot(im, w0i_ref[...], preferred_element_type=f32)
    h = jnp.maximum(h + b0_ref[...], 0.0)
    h = jnp.maximum(jnp.dot(h, w1_ref[...], preferred_element_type=f32)
                    + b1_ref[...], 0.0)
    h = jnp.maximum(jnp.dot(h, w2_ref[...], preferred_element_type=f32)
                    + b2_ref[...], 0.0)
    gmf = ug * ig
    o_ref[...] = (jnp.dot(gmf, whg_ref[...], preferred_element_type=f32)
                  + jnp.dot(h, whh_ref[...], preferred_element_type=f32)
                  + bh_ref[...])


def _tc_dense(us, is_, ug, ig, um, im, W0, b0, W1, b1, W2, b2, Wh, bh):
    w0u = W0[:, :DIM].T
    w0i = W0[:, DIM:].T
    w1 = W1.T
    w2 = W2.T
    whg = Wh[:, :DIM].T
    whh = Wh[:, DIM:].T
    b0r = b0.reshape(1, -1)
    b1r = b1.reshape(1, -1)
    b2r = b2.reshape(1, -1)
    bhr = bh.reshape(1, 1)

    n_blk = BATCH // BLK
    row_spec = pl.BlockSpec((BLK, 128), lambda b: (b, 0))
    sel_spec = pl.BlockSpec((BLK, 1), lambda b: (b, 0))

    def w_spec(shape):
        return pl.BlockSpec(shape, lambda b: (0, 0))

    out = pl.pallas_call(
        _tc_body,
        grid=(n_blk,),
        in_specs=[
            sel_spec, sel_spec,
            row_spec, row_spec, row_spec, row_spec,
            w_spec(w0u.shape), w_spec(w0i.shape), w_spec(b0r.shape),
            w_spec(w1.shape), w_spec(b1r.shape),
            w_spec(w2.shape), w_spec(b2r.shape),
            w_spec(whg.shape), w_spec(whh.shape), w_spec(bhr.shape),
        ],
        out_specs=pl.BlockSpec((BLK, 1), lambda b: (b, 0)),
        out_shape=jax.ShapeDtypeStruct((BATCH, 1), jnp.float32),
    )(us, is_, ug, ig, um, im, w0u, w0i, b0r, w1, b1r, w2, b2r,
      whg, whh, bhr)
    return out[:, 0]


def kernel(u, i, user_gmf, item_gmf, user_mlp, item_mlp,
           W0, b0, W1, b1, W2, b2, Wh, bh):
    u2 = (((u >> 9) << 7) | (u & 127)).reshape(BATCH // 128, 128)
    i2 = (((i >> 9) << 7) | (i & 127)).reshape(BATCH // 128, 128)
    um4, im4 = _relayout2(user_mlp.T, item_mlp.T)
    um, im = _sc_gather2(u2, i2, um4, im4)
    ug4, ig4 = _relayout2(user_gmf.T, item_gmf.T)
    ug, ig = _sc_gather2(u2, i2, ug4, ig4)
    us = ((u >> 7) & 3).reshape(BATCH, 1)
    is_ = ((i >> 7) & 3).reshape(BATCH, 1)
    return _tc_dense(us, is_, ug, ig, um, im,
                     W0, b0, W1, b1, W2, b2, Wh, bh)


# speedup vs baseline: 6.2499x; 1.0248x over previous
import functools

import jax
import jax.numpy as jnp
from jax import lax
from jax.experimental import pallas as pl
from jax.experimental.pallas import tpu as pltpu
from jax.experimental.pallas import tpu_sc as plsc

NC = 2
NS = 16
NW = NC * NS

BATCH = 16384
DIM = 32
NROWS = 1000000
CHUNK = 128
CHUNKS_PER_W = BATCH // (NW * CHUNK)

LB = 32768
NG = (NROWS + LB - 1) // LB
OUTR = NG * (LB // 4)


def _repack_one(x, o_ref):
    for s in range(LB // 512):
        stacked = jnp.concatenate(
            [x[:, 512 * s + 128 * q:512 * s + 128 * (q + 1)]
             for q in range(4)], axis=0)
        o_ref[128 * s:128 * (s + 1), :] = stacked.T


def _relayout_body(x1, x2, o1, o2):
    _repack_one(x1[...], o1)
    _repack_one(x2[...], o2)


def _relayout2(t1, t2):
    in_spec = pl.BlockSpec((DIM, LB), lambda g: (0, g))
    out_spec = pl.BlockSpec((LB // 4, 128), lambda g: (g, 0))
    out_t = jax.ShapeDtypeStruct((OUTR, 128), jnp.float32)
    return pl.pallas_call(
        _relayout_body,
        grid=(NG,),
        in_specs=[in_spec] * 2,
        out_specs=[out_spec] * 2,
        out_shape=[out_t] * 2,
    )(t1, t2)


def _sc_gather2(u2, i2, tblA, tblB):
    mesh = plsc.VectorSubcoreMesh(core_axis_name="c", subcore_axis_name="s")
    out_t = jax.ShapeDtypeStruct((BATCH, 128), jnp.float32)

    @functools.partial(
        pl.kernel,
        out_type=(out_t, out_t),
        mesh=mesh,
        scratch_types=[
            pltpu.VMEM((CHUNKS_PER_W, 128), jnp.int32),
            pltpu.VMEM((CHUNKS_PER_W, 128), jnp.int32),
            pltpu.VMEM((CHUNK, 128), jnp.float32),
            pltpu.VMEM((CHUNK, 128), jnp.float32),
            pltpu.VMEM((CHUNK, 128), jnp.float32),
            pltpu.VMEM((CHUNK, 128), jnp.float32),
            pltpu.SemaphoreType.DMA,
            pltpu.SemaphoreType.DMA,
            pltpu.SemaphoreType.DMA,
            pltpu.SemaphoreType.DMA,
        ],
    )
    def k(tblA_hbm, tblB_hbm, u_hbm, i_hbm, outA, outB,
          uidx, iidx, bufA0, bufA1, bufB0, bufB1, sA0, sA1, sB0, sB1):
        wid = lax.axis_index("s") * NC + lax.axis_index("c")
        row0 = wid * CHUNKS_PER_W
        pltpu.sync_copy(u_hbm.at[pl.ds(row0, CHUNKS_PER_W)], uidx)
        pltpu.sync_copy(i_hbm.at[pl.ds(row0, CHUNKS_PER_W)], iidx)
        bufsA = (bufA0, bufA1)
        bufsB = (bufB0, bufB1)
        semsA = (sA0, sA1)
        semsB = (sB0, sB1)
        descs = [None, None]

        def fire(j):
            p = j % 2
            dA = pltpu.async_copy(tblA_hbm.at[uidx.at[j]], bufsA[p], semsA[p])
            dB = pltpu.async_copy(tblB_hbm.at[iidx.at[j]], bufsB[p], semsB[p])
            descs[p] = (dA, dB)

        def drain(j):
            p = j % 2
            base = (row0 + j) * CHUNK
            descs[p][0].wait()
            pltpu.sync_copy(bufsA[p], outA.at[pl.ds(base, CHUNK)])
            descs[p][1].wait()
            pltpu.sync_copy(bufsB[p], outB.at[pl.ds(base, CHUNK)])

        fire(0)
        for j in range(1, CHUNKS_PER_W):
            fire(j)
            drain(j - 1)
        drain(CHUNKS_PER_W - 1)

    return k(tblA, tblB, u2, i2)


BLK = 4096


def _sel(x, m):
    r = jnp.where(m == 0, x[:, 0 * DIM:1 * DIM], x[:, 1 * DIM:2 * DIM])
    r = jnp.where(m == 2, x[:, 2 * DIM:3 * DIM], r)
    return jnp.where(m == 3, x[:, 3 * DIM:4 * DIM], r)


def _tc_body(us_ref, is_ref, ug_ref, ig_ref, um_ref, im_ref,
             w0u_ref, w0i_ref, b0_ref, w1_ref, b1_ref, w2_ref, b2_ref,
             whg_ref, whh_ref, bh_ref, o_ref):
    f32 = jnp.float32
    mu = us_ref[...]
    mi = is_ref[...]
    ug = _sel(ug_ref[...], mu)
    ig = _sel(ig_ref[...], mi)
    um = _sel(um_ref[...], mu)
    im = _sel(im_ref[...], mi)
    h = jnp.dot(um, w0u_ref[...], preferred_element_type=f32)
    h = h + jnp.dot(im, w0i_ref[...], preferred_element_type=f32)
    h = jnp.maximum(h + b0_ref[...], 0.0)
    h = jnp.maximum(jnp.dot(h, w1_ref[...], preferred_element_type=f32)
                    + b1_ref[...], 0.0)
    h = jnp.maximum(jnp.dot(h, w2_ref[...], preferred_element_type=f32)
                    + b2_ref[...], 0.0)
    gmf = ug * ig
    o_ref[...] = (jnp.dot(gmf, whg_ref[...], preferred_element_type=f32)
                  + jnp.dot(h, whh_ref[...], preferred_element_type=f32)
                  + bh_ref[...])


def _tc_dense(us, is_, ug, ig, um, im, W0, b0, W1, b1, W2, b2, Wh, bh):
    w0u = W0[:, :DIM].T
    w0i = W0[:, DIM:].T
    w1 = W1.T
    w2 = W2.T
    whg = Wh[:, :DIM].T
    whh = Wh[:, DIM:].T
    b0r = b0.reshape(1, -1)
    b1r = b1.reshape(1, -1)
    b2r = b2.reshape(1, -1)
    bhr = bh.reshape(1, 1)

    n_blk = BATCH // BLK
    row_spec = pl.BlockSpec((BLK, 128), lambda b: (b, 0))
    sel_spec = pl.BlockSpec((BLK, 1), lambda b: (b, 0))

    def w_spec(shape):
        return pl.BlockSpec(shape, lambda b: (0, 0))

    out = pl.pallas_call(
        _tc_body,
        grid=(n_blk,),
        in_specs=[
            sel_spec, sel_spec,
            row_spec, row_spec, row_spec, row_spec,
            w_spec(w0u.shape), w_spec(w0i.shape), w_spec(b0r.shape),
            w_spec(w1.shape), w_spec(b1r.shape),
            w_spec(w2.shape), w_spec(b2r.shape),
            w_spec(whg.shape), w_spec(whh.shape), w_spec(bhr.shape),
        ],
        out_specs=pl.BlockSpec((BLK, 1), lambda b: (b, 0)),
        out_shape=jax.ShapeDtypeStruct((BATCH, 1), jnp.float32),
    )(us, is_, ug, ig, um, im, w0u, w0i, b0r, w1, b1r, w2, b2r,
      whg, whh, bhr)
    return out[:, 0]


def kernel(u, i, user_gmf, item_gmf, user_mlp, item_mlp,
           W0, b0, W1, b1, W2, b2, Wh, bh):
    u2 = (((u >> 9) << 7) | (u & 127)).reshape(BATCH // 128, 128)
    i2 = (((i >> 9) << 7) | (i & 127)).reshape(BATCH // 128, 128)
    um4, im4 = _relayout2(user_mlp.T, item_mlp.T)
    um, im = _sc_gather2(u2, i2, um4, im4)
    ug4, ig4 = _relayout2(user_gmf.T, item_gmf.T)
    ug, ig = _sc_gather2(u2, i2, ug4, ig4)
    us = ((u >> 7) & 3).reshape(BATCH, 1)
    is_ = ((i >> 7) & 3).reshape(BATCH, 1)
    return _tc_dense(us, is_, ug, ig, um, im,
                     W0, b0, W1, b1, W2, b2, Wh, bh)
